# Initial kernel scaffold; baseline (speedup 1.0000x reference)
import jax
import jax.numpy as jnp
from jax.experimental import pallas as pl


def kernel(x, emb_table, pos_table):
    raise NotImplementedError("write your pallas kernel here")


# speedup vs baseline: 2.2843x; 2.2843x over previous
import functools

import jax
import jax.numpy as jnp
from jax import lax
from jax.experimental import pallas as pl
from jax.experimental.pallas import tpu as pltpu
from jax.experimental.pallas import tpu_sc as plsc

B = 4096
S = 200
H = 64
NC = 2
NS = 16
NW = NC * NS
ROWS = B * S
SEQ_PER_W = B // NW


def _body(x_hbm, emb_hbm, pos_hbm, out_hbm, idx_v, rows_v, pos_v, sem):
    wid = lax.axis_index("s") * NC + lax.axis_index("c")

    pltpu.sync_copy(pos_hbm, pos_v)

    def seq_step(i, carry):
        base = (wid * SEQ_PER_W + i) * S
        pltpu.sync_copy(x_hbm.at[pl.ds(base, S)], idx_v)
        cp1 = pltpu.async_copy(emb_hbm.at[idx_v.at[pl.ds(0, 128)]],
                               rows_v.at[pl.ds(0, 128)], sem)
        cp2 = pltpu.async_copy(emb_hbm.at[idx_v.at[pl.ds(128, S - 128)]],
                               rows_v.at[pl.ds(128, S - 128)], sem)
        cp1.wait()
        cp2.wait()

        def add_step(r, carry2):
            for j in range(H // 16):
                sl = pl.ds(j * 16, 16)
                rows_v[r, sl] = rows_v[r, sl] + pos_v[r, sl]
            return carry2

        lax.fori_loop(0, S, add_step, 0, unroll=4)
        pltpu.sync_copy(rows_v, out_hbm.at[pl.ds(base, S)])
        return carry

    lax.fori_loop(0, SEQ_PER_W, seq_step, 0)


@jax.jit
def _embed(x_flat, emb_table, pos_table):
    mesh = plsc.VectorSubcoreMesh(core_axis_name="c", subcore_axis_name="s",
                                  num_cores=NC, num_subcores=NS)
    run = pl.kernel(
        _body,
        out_type=jax.ShapeDtypeStruct((ROWS, H), jnp.float32),
        mesh=mesh,
        scratch_types=[
            pltpu.VMEM((S,), jnp.int32),
            pltpu.VMEM((S, H), jnp.float32),
            pltpu.VMEM((S, H), jnp.float32),
            pltpu.SemaphoreType.DMA,
        ],
        compiler_params=pltpu.CompilerParams(use_tc_tiling_on_sc=False),
    )
    return run(x_flat, emb_table, pos_table)


def kernel(x, emb_table, pos_table):
    x_flat = x.reshape(ROWS).astype(jnp.int32)
    out = _embed(x_flat, emb_table, pos_table)
    return out.reshape(B, S, H)


# speedup vs baseline: 4.2188x; 1.8469x over previous
import functools

import jax
import jax.numpy as jnp
from jax import lax
from jax.experimental import pallas as pl
from jax.experimental.pallas import tpu as pltpu
from jax.experimental.pallas import tpu_sc as plsc

B = 4096
S = 200
H = 64
NC = 2
NS = 16
NW = NC * NS
ROWS = B * S
NSEQ = B // NW
WROWS = NSEQ * S
RING = 4
C0 = 128
C1 = S - C0


def _body(x_hbm, emb_hbm, pos_hbm, out_hbm, idx_all, rows, pos_v, *sems):
    gat_sems = sems[:RING]
    out_sems = sems[RING:]
    wid = lax.axis_index("s") * NC + lax.axis_index("c")
    wbase = wid * WROWS

    pltpu.sync_copy(x_hbm.at[pl.ds(wbase, WROWS)], idx_all)
    pltpu.sync_copy(pos_hbm, pos_v)

    def gather_descs(s, b):
        return (
            pltpu.make_async_copy(
                emb_hbm.at[idx_all.at[pl.ds(s * S, C0)]],
                rows.at[b, pl.ds(0, C0)], gat_sems[b]),
            pltpu.make_async_copy(
                emb_hbm.at[idx_all.at[pl.ds(s * S + C0, C1)]],
                rows.at[b, pl.ds(C0, C1)], gat_sems[b]),
        )

    def out_desc(s, b):
        return pltpu.make_async_copy(
            rows.at[b], out_hbm.at[pl.ds(wbase + s * S, S)], out_sems[b])

    def fire_gather(s, b):
        for d in gather_descs(s, b):
            d.start()

    def wait_gather(s, b):
        for d in gather_descs(s, b):
            d.wait()

    def add_pos(b):
        @plsc.parallel_loop(0, S, 1, unroll=4)
        def _(r):
            for j in range(H // 16):
                sl = pl.ds(j * 16, 16)
                rows[b, r, sl] = rows[b, r, sl] + pos_v[r, sl]

    def step(s, b, do_wait_out=True, do_fire=True):
        if do_wait_out:
            out_desc(s - 2, (b + 2) % RING).wait()
        if do_fire:
            fire_gather(s + 2, (b + 2) % RING)
        wait_gather(s, b)
        add_pos(b)
        out_desc(s, b).start()

    fire_gather(0, 0)
    fire_gather(1, 1)
    step(0, 0, do_wait_out=False)
    step(1, 1, do_wait_out=False)
    step(2, 2)
    step(3, 3)

    def steady(g, carry):
        s0 = 4 + g * RING
        for b in range(RING):
            step(s0 + b, b)
        return carry

    lax.fori_loop(0, (NSEQ - 8) // RING, steady, 0)

    step(NSEQ - 4, 0)
    step(NSEQ - 3, 1)
    step(NSEQ - 2, 2, do_fire=False)
    step(NSEQ - 1, 3, do_fire=False)
    out_desc(NSEQ - 2, 2).wait()
    out_desc(NSEQ - 1, 3).wait()


@jax.jit
def _embed(x_flat, emb_table, pos_table):
    mesh = plsc.VectorSubcoreMesh(core_axis_name="c", subcore_axis_name="s",
                                  num_cores=NC, num_subcores=NS)
    run = pl.kernel(
        _body,
        out_type=jax.ShapeDtypeStruct((ROWS, H), jnp.float32),
        mesh=mesh,
        scratch_types=[
            pltpu.VMEM((WROWS,), jnp.int32),
            pltpu.VMEM((RING, S, H), jnp.float32),
            pltpu.VMEM((S, H), jnp.float32),
        ] + [pltpu.SemaphoreType.DMA] * (2 * RING),
        compiler_params=pltpu.CompilerParams(use_tc_tiling_on_sc=False),
    )
    return run(x_flat, emb_table, pos_table)


def kernel(x, emb_table, pos_table):
    x_flat = x.reshape(ROWS).astype(jnp.int32)
    out = _embed(x_flat, emb_table, pos_table)
    return out.reshape(B, S, H)


# speedup vs baseline: 4.2257x; 1.0016x over previous
import jax
import jax.numpy as jnp
from jax import lax
from jax.experimental import pallas as pl
from jax.experimental.pallas import tpu as pltpu
from jax.experimental.pallas import tpu_sc as plsc

B = 4096
S = 200
H = 64
NC = 2
NS = 16
NW = NC * NS
NSEQ = B // NW
RING = 4
C0 = 128
C1 = S - C0


def _body(x_hbm, emb_hbm, pos_hbm, out_hbm, idx_all, rows, pos_v, *sems):
    gat_sems = sems[:RING]
    out_sems = sems[RING:]
    wid = lax.axis_index("s") * NC + lax.axis_index("c")
    wseq = wid * NSEQ

    pltpu.sync_copy(x_hbm.at[pl.ds(wseq, NSEQ)], idx_all)
    pltpu.sync_copy(pos_hbm, pos_v)

    def gather_descs(s, b):
        return (
            pltpu.make_async_copy(
                emb_hbm.at[idx_all.at[s, pl.ds(0, C0)]],
                rows.at[b, pl.ds(0, C0)], gat_sems[b]),
            pltpu.make_async_copy(
                emb_hbm.at[idx_all.at[s, pl.ds(C0, C1)]],
                rows.at[b, pl.ds(C0, C1)], gat_sems[b]),
        )

    def out_desc(s, b):
        return pltpu.make_async_copy(
            rows.at[b], out_hbm.at[wseq + s], out_sems[b])

    def fire_gather(s, b):
        for d in gather_descs(s, b):
            d.start()

    def wait_gather(s, b):
        for d in gather_descs(s, b):
            d.wait()

    def add_pos(b):
        @plsc.parallel_loop(0, S, 1, unroll=4)
        def _(r):
            for j in range(H // 16):
                sl = pl.ds(j * 16, 16)
                rows[b, r, sl] = rows[b, r, sl] + pos_v[r, sl]

    def step(s, b, do_wait_out=True, do_fire=True):
        if do_wait_out:
            out_desc(s - 2, (b + 2) % RING).wait()
        if do_fire:
            fire_gather(s + 2, (b + 2) % RING)
        wait_gather(s, b)
        add_pos(b)
        out_desc(s, b).start()

    fire_gather(0, 0)
    fire_gather(1, 1)
    step(0, 0, do_wait_out=False)
    step(1, 1, do_wait_out=False)
    step(2, 2)
    step(3, 3)

    def steady(g, carry):
        s0 = 4 + g * RING
        for b in range(RING):
            step(s0 + b, b)
        return carry

    lax.fori_loop(0, (NSEQ - 8) // RING, steady, 0)

    step(NSEQ - 4, 0)
    step(NSEQ - 3, 1)
    step(NSEQ - 2, 2, do_fire=False)
    step(NSEQ - 1, 3, do_fire=False)
    out_desc(NSEQ - 2, 2).wait()
    out_desc(NSEQ - 1, 3).wait()


@jax.jit
def _embed(x, emb_table, pos_table):
    mesh = plsc.VectorSubcoreMesh(core_axis_name="c", subcore_axis_name="s",
                                  num_cores=NC, num_subcores=NS)
    run = pl.kernel(
        _body,
        out_type=jax.ShapeDtypeStruct((B, S, H), jnp.float32),
        mesh=mesh,
        scratch_types=[
            pltpu.VMEM((NSEQ, S), jnp.int32),
            pltpu.VMEM((RING, S, H), jnp.float32),
            pltpu.VMEM((S, H), jnp.float32),
        ] + [pltpu.SemaphoreType.DMA] * (2 * RING),
        compiler_params=pltpu.CompilerParams(use_tc_tiling_on_sc=False),
    )
    return run(x, emb_table, pos_table)


def kernel(x, emb_table, pos_table):
    return _embed(x.astype(jnp.int32), emb_table, pos_table)
